# baseline (device time: 981099 ns/iter reference)
import jax
import jax.numpy as jnp
from jax import lax
from jax.experimental import pallas as pl
from jax.experimental.pallas import tpu as pltpu

N_DEV = 4
T = 512
D = 512
F = 1024
E_LOC = 2
C = 160


def kernel(x, assign, W1, W2):
    dest = assign // E_LOC
    idx = []
    valid = []
    for j in range(N_DEV):
        order = jnp.argsort((dest != j).astype(jnp.int32), stable=True)
        ij = order[:C]
        idx.append(ij)
        valid.append(dest[ij] == j)
    idx = jnp.stack(idx)
    valid = jnp.stack(valid)

    x_pack = x[idx.reshape(-1)].astype(jnp.bfloat16)
    a_pack = jnp.where(valid.reshape(-1), assign[idx.reshape(-1)],
                       -1).astype(jnp.int32).reshape(N_DEV * C, 1)
    W1b = W1.astype(jnp.bfloat16)
    W2b = W2.astype(jnp.bfloat16)

    def body(xp_ref, a_ref, w1_ref, w2_ref, out_ref,
             xrx, arx, yloc, yrx,
             x_send, x_recv, a_send, a_recv, y_send, y_recv):
        my = lax.axis_index("i")

        barrier = pltpu.get_barrier_semaphore()
        for r in range(1, N_DEV):
            tgt = lax.rem(my + r, N_DEV)
            pl.semaphore_signal(barrier, inc=1, device_id=(tgt,),
                                device_id_type=pl.DeviceIdType.MESH)
        pl.semaphore_wait(barrier, N_DEV - 1)

        def compute_chunk(xa, aa):
            acc = jnp.zeros((C, D), jnp.float32)
            for k in range(E_LOC):
                eid = my * E_LOC + k
                xm = jnp.where(aa == eid, xa,
                               jnp.bfloat16(0.0)).astype(jnp.bfloat16)
                h = jnp.maximum(
                    jnp.dot(xm, w1_ref[k],
                            preferred_element_type=jnp.float32),
                    0.0).astype(jnp.bfloat16)
                acc = acc + jnp.dot(h, w2_ref[k],
                                    preferred_element_type=jnp.float32)
            return acc.astype(jnp.bfloat16)

        drain = []
        for r in range(1, N_DEV):
            tgt = lax.rem(my + r, N_DEV)
            xs = pltpu.make_async_remote_copy(
                src_ref=xp_ref.at[pl.ds(tgt * C, C), :],
                dst_ref=xrx.at[pl.ds(my * C, C), :],
                send_sem=x_send.at[r - 1],
                recv_sem=x_recv.at[r - 1],
                device_id=(tgt,),
                device_id_type=pl.DeviceIdType.MESH,
            )
            sa = pltpu.make_async_remote_copy(
                src_ref=a_ref.at[pl.ds(tgt * C, C), :],
                dst_ref=arx.at[pl.ds(my * C, C), :],
                send_sem=a_send.at[r - 1],
                recv_sem=a_recv.at[r - 1],
                device_id=(tgt,),
                device_id_type=pl.DeviceIdType.MESH,
            )
            xs.start()
            sa.start()
            drain += [xs, sa]

        yrx[pl.ds(my * C, C), :] = compute_chunk(
            xp_ref[pl.ds(my * C, C), :], a_ref[pl.ds(my * C, C), :])

        for r in range(1, N_DEV):
            src_pos = lax.rem(my - r + N_DEV, N_DEV)
            pltpu.make_async_remote_copy(
                src_ref=xp_ref.at[pl.ds(0, C), :],
                dst_ref=xrx.at[pl.ds(src_pos * C, C), :],
                send_sem=x_send.at[r - 1], recv_sem=x_recv.at[r - 1],
                device_id=(0,), device_id_type=pl.DeviceIdType.MESH,
            ).wait_recv()
            pltpu.make_async_remote_copy(
                src_ref=a_ref.at[pl.ds(0, C), :],
                dst_ref=arx.at[pl.ds(src_pos * C, C), :],
                send_sem=a_send.at[r - 1], recv_sem=a_recv.at[r - 1],
                device_id=(0,), device_id_type=pl.DeviceIdType.MESH,
            ).wait_recv()

            yloc[pl.ds(src_pos * C, C), :] = compute_chunk(
                xrx[pl.ds(src_pos * C, C), :],
                arx[pl.ds(src_pos * C, C), :])
            yd = pltpu.make_async_remote_copy(
                src_ref=yloc.at[pl.ds(src_pos * C, C), :],
                dst_ref=yrx.at[pl.ds(my * C, C), :],
                send_sem=y_send.at[r - 1],
                recv_sem=y_recv.at[(N_DEV - r) - 1],
                device_id=(src_pos,),
                device_id_type=pl.DeviceIdType.MESH,
            )
            yd.start()
            drain.append(yd)

        for rc in range(1, N_DEV):
            src_pos = lax.rem(my - rc + N_DEV, N_DEV)
            pltpu.make_async_remote_copy(
                src_ref=yloc.at[pl.ds(0, C), :],
                dst_ref=yrx.at[pl.ds(src_pos * C, C), :],
                send_sem=y_send.at[rc - 1],
                recv_sem=y_recv.at[rc - 1],
                device_id=(0,),
                device_id_type=pl.DeviceIdType.MESH,
            ).wait_recv()

        out_ref[...] = yrx[...]

        for d in drain:
            d.wait_send()

    y = pl.pallas_call(
        body,
        out_shape=jax.ShapeDtypeStruct((N_DEV * C, D), jnp.bfloat16),
        in_specs=[pl.BlockSpec(memory_space=pltpu.VMEM)] * 4,
        out_specs=pl.BlockSpec(memory_space=pltpu.VMEM),
        scratch_shapes=[
            pltpu.VMEM((N_DEV * C, D), jnp.bfloat16),
            pltpu.VMEM((N_DEV * C, 1), jnp.int32),
            pltpu.VMEM((N_DEV * C, D), jnp.bfloat16),
            pltpu.VMEM((N_DEV * C, D), jnp.bfloat16),
            pltpu.SemaphoreType.DMA((N_DEV - 1,)),
            pltpu.SemaphoreType.DMA((N_DEV - 1,)),
            pltpu.SemaphoreType.DMA((N_DEV - 1,)),
            pltpu.SemaphoreType.DMA((N_DEV - 1,)),
            pltpu.SemaphoreType.DMA((N_DEV - 1,)),
            pltpu.SemaphoreType.DMA((N_DEV - 1,)),
        ],
        compiler_params=pltpu.CompilerParams(collective_id=0),
    )(x_pack, a_pack, W1b, W2b)

    vals = jnp.where(valid.reshape(-1)[:, None], y.astype(jnp.float32), 0.0)
    return jnp.zeros((T, D), jnp.float32).at[idx.reshape(-1)].add(vals)


# device time: 25401 ns/iter; 38.6244x vs baseline; 38.6244x over previous
import jax
import jax.numpy as jnp
from jax import lax
from jax.experimental import pallas as pl
from jax.experimental.pallas import tpu as pltpu

N_DEV = 4
T = 512
D = 512
F = 1024
E_LOC = 2
C = 160


def kernel(x, assign, W1, W2):
    dest = assign // E_LOC
    onehot = (dest[:, None] == jnp.arange(N_DEV)[None, :])
    rank = jnp.cumsum(onehot.astype(jnp.int32), axis=0) - 1
    myrank = jnp.sum(jnp.where(onehot, rank, 0), axis=1)
    slot = (dest * C + myrank).astype(jnp.int32)

    slot_row = slot.reshape(1, T)
    a1_col = (assign + 1).astype(jnp.float32).reshape(T, 1)
    xb = x.astype(jnp.bfloat16)
    W1b = W1.astype(jnp.bfloat16)
    W2b = W2.astype(jnp.bfloat16)

    def body(x_ref, slot_ref, a1_ref, w1_ref, w2_ref, out_ref,
             xpk, apk, xrx, arx, yloc, yrx,
             x_send, x_recv, a_send, a_recv, y_send, y_recv):
        my = lax.axis_index("i")

        barrier = pltpu.get_barrier_semaphore()
        for r in range(1, N_DEV):
            tgt = lax.rem(my + r, N_DEV)
            pl.semaphore_signal(barrier, inc=1, device_id=(tgt,),
                                device_id_type=pl.DeviceIdType.MESH)
        pl.semaphore_wait(barrier, N_DEV - 1)

        rows = lax.broadcasted_iota(jnp.int32, (N_DEV * C, T), 0)
        P = (rows == slot_ref[...]).astype(jnp.bfloat16)
        xpk[...] = jnp.dot(P, x_ref[...],
                           preferred_element_type=jnp.float32
                           ).astype(jnp.bfloat16)
        apk[...] = jnp.dot(P, a1_ref[...].astype(jnp.bfloat16),
                           preferred_element_type=jnp.float32
                           ).astype(jnp.bfloat16)

        def compute_chunk(xa, aa):
            acc = jnp.zeros((C, D), jnp.float32)
            for k in range(E_LOC):
                eid1 = (my * E_LOC + k + 1).astype(jnp.bfloat16)
                xm = jnp.where(aa == eid1, xa,
                               jnp.bfloat16(0.0)).astype(jnp.bfloat16)
                h = jnp.maximum(
                    jnp.dot(xm, w1_ref[k],
                            preferred_element_type=jnp.float32),
                    0.0).astype(jnp.bfloat16)
                acc = acc + jnp.dot(h, w2_ref[k],
                                    preferred_element_type=jnp.float32)
            return acc.astype(jnp.bfloat16)

        drain = []
        for r in range(1, N_DEV):
            tgt = lax.rem(my + r, N_DEV)
            xs = pltpu.make_async_remote_copy(
                src_ref=xpk.at[pl.ds(tgt * C, C), :],
                dst_ref=xrx.at[pl.ds(my * C, C), :],
                send_sem=x_send.at[r - 1],
                recv_sem=x_recv.at[r - 1],
                device_id=(tgt,),
                device_id_type=pl.DeviceIdType.MESH,
            )
            sa = pltpu.make_async_remote_copy(
                src_ref=apk.at[pl.ds(tgt * C, C), :],
                dst_ref=arx.at[pl.ds(my * C, C), :],
                send_sem=a_send.at[r - 1],
                recv_sem=a_recv.at[r - 1],
                device_id=(tgt,),
                device_id_type=pl.DeviceIdType.MESH,
            )
            xs.start()
            sa.start()
            drain += [xs, sa]

        yrx[pl.ds(my * C, C), :] = compute_chunk(
            xpk[pl.ds(my * C, C), :], apk[pl.ds(my * C, C), :])

        for r in range(1, N_DEV):
            src_pos = lax.rem(my - r + N_DEV, N_DEV)
            pltpu.make_async_remote_copy(
                src_ref=xpk.at[pl.ds(0, C), :],
                dst_ref=xrx.at[pl.ds(src_pos * C, C), :],
                send_sem=x_send.at[r - 1], recv_sem=x_recv.at[r - 1],
                device_id=(0,), device_id_type=pl.DeviceIdType.MESH,
            ).wait_recv()
            pltpu.make_async_remote_copy(
                src_ref=apk.at[pl.ds(0, C), :],
                dst_ref=arx.at[pl.ds(src_pos * C, C), :],
                send_sem=a_send.at[r - 1], recv_sem=a_recv.at[r - 1],
                device_id=(0,), device_id_type=pl.DeviceIdType.MESH,
            ).wait_recv()

            yloc[pl.ds(src_pos * C, C), :] = compute_chunk(
                xrx[pl.ds(src_pos * C, C), :],
                arx[pl.ds(src_pos * C, C), :])
            yd = pltpu.make_async_remote_copy(
                src_ref=yloc.at[pl.ds(src_pos * C, C), :],
                dst_ref=yrx.at[pl.ds(my * C, C), :],
                send_sem=y_send.at[r - 1],
                recv_sem=y_recv.at[(N_DEV - r) - 1],
                device_id=(src_pos,),
                device_id_type=pl.DeviceIdType.MESH,
            )
            yd.start()
            drain.append(yd)

        for rc in range(1, N_DEV):
            src_pos = lax.rem(my - rc + N_DEV, N_DEV)
            pltpu.make_async_remote_copy(
                src_ref=yloc.at[pl.ds(0, C), :],
                dst_ref=yrx.at[pl.ds(src_pos * C, C), :],
                send_sem=y_send.at[rc - 1],
                recv_sem=y_recv.at[rc - 1],
                device_id=(0,),
                device_id_type=pl.DeviceIdType.MESH,
            ).wait_recv()

        out_ref[...] = lax.dot_general(
            P, yrx[...], dimension_numbers=(((0,), (0,)), ((), ())),
            preferred_element_type=jnp.float32)

        for d in drain:
            d.wait_send()

    return pl.pallas_call(
        body,
        out_shape=jax.ShapeDtypeStruct((T, D), jnp.float32),
        in_specs=[pl.BlockSpec(memory_space=pltpu.VMEM)] * 5,
        out_specs=pl.BlockSpec(memory_space=pltpu.VMEM),
        scratch_shapes=[
            pltpu.VMEM((N_DEV * C, D), jnp.bfloat16),
            pltpu.VMEM((N_DEV * C, 1), jnp.bfloat16),
            pltpu.VMEM((N_DEV * C, D), jnp.bfloat16),
            pltpu.VMEM((N_DEV * C, 1), jnp.bfloat16),
            pltpu.VMEM((N_DEV * C, D), jnp.bfloat16),
            pltpu.VMEM((N_DEV * C, D), jnp.bfloat16),
            pltpu.SemaphoreType.DMA((N_DEV - 1,)),
            pltpu.SemaphoreType.DMA((N_DEV - 1,)),
            pltpu.SemaphoreType.DMA((N_DEV - 1,)),
            pltpu.SemaphoreType.DMA((N_DEV - 1,)),
            pltpu.SemaphoreType.DMA((N_DEV - 1,)),
            pltpu.SemaphoreType.DMA((N_DEV - 1,)),
        ],
        compiler_params=pltpu.CompilerParams(collective_id=0),
    )(xb, slot_row, a1_col, W1b, W2b)
